# baseline (speedup 1.0000x reference)
import functools
import math

import jax
import jax.numpy as jnp
from jax import lax
from jax.experimental import pallas as pl
from jax.experimental.pallas import tpu as pltpu
from jax.experimental.pallas import tpu_sc as plsc

D = 64
LANES = 128
NBUF = 4
NC, NS = 2, 16
NW = NC * NS
SCALE = math.sqrt(D)


def _sc_embed(xT, tableP, T, B):
    mesh = plsc.VectorSubcoreMesh(core_axis_name="c", subcore_axis_name="s")

    @functools.partial(
        pl.kernel,
        mesh=mesh,
        out_type=jax.ShapeDtypeStruct((T, D, B), jnp.float32),
        scratch_types=[
            pltpu.VMEM((T, LANES), jnp.int32),
            [pltpu.VMEM((LANES, LANES), jnp.float32) for _ in range(NBUF)],
            [pltpu.VMEM((D, LANES), jnp.float32) for _ in range(NBUF)],
            [pltpu.VMEM((LANES,), jnp.int32) for _ in range(NBUF)],
            [pltpu.VMEM((LANES,), jnp.int32) for _ in range(NBUF)],
            [pltpu.SemaphoreType.DMA for _ in range(NBUF)],
            [pltpu.SemaphoreType.DMA for _ in range(NBUF)],
        ],
        compiler_params=pltpu.CompilerParams(
            use_tc_tiling_on_sc=True, needs_layout_passes=False),
    )
    def k(x_hbm, table_hbm, out_hbm, idxv, rows, obuf, gidx, hbuf, gsem, osem):
        wid = lax.axis_index("s") * NC + lax.axis_index("c")
        b0 = wid * LANES
        iota = lax.iota(jnp.int32, 16)

        def prep(t, s):
            for c in range(LANES // 16):
                v = idxv[t, pl.ds(c * 16, 16)]
                gidx[s][pl.ds(c * 16, 16)] = v >> 1
                hbuf[s][pl.ds(c * 16, 16)] = (v & 1) << 6

        def gfire(s):
            pltpu.async_copy(table_hbm.at[gidx[s]], rows[s], gsem[s])

        def gwait(s):
            pltpu.make_async_copy(
                table_hbm.at[pl.ds(0, LANES)], rows[s], gsem[s]).wait()

        def ofire(t, s):
            pltpu.async_copy(
                obuf[s], out_hbm.at[t, :, pl.ds(b0, LANES)], osem[s])

        def owait(s):
            pltpu.make_async_copy(
                out_hbm.at[0, :, pl.ds(0, LANES)], obuf[s], osem[s]).wait()

        def transpose_scale(s):
            for h in range(LANES // 16):

                @plsc.parallel_loop(0, 16)
                def _(d):
                    perm = (iota + d) & 15
                    rd = perm + 16 * h
                    hv_d = plsc.load_gather(hbuf[s], [rd])
                    colb = hv_d + iota
                    for g in range(D // 16):
                        val = plsc.load_gather(rows[s], [rd, colb + 16 * g])
                        plsc.store_scatter(
                            obuf[s], [iota + 16 * g, rd], val * SCALE)

        def body(g, s):
            s1 = (s + 1) % NBUF

            @pl.when(g >= NBUF - 1)
            def _():
                owait(s1)

            @pl.when(g + 1 < T)
            def _():
                prep(g + 1, s1)
                gfire(s1)

            gwait(s)
            transpose_scale(s)
            ofire(g, s)

        pltpu.sync_copy(x_hbm.at[:, pl.ds(b0, LANES)], idxv)
        prep(0, 0)
        gfire(0)

        @pl.loop(0, T, step=NBUF)
        def _(p):
            for b in range(NBUF):
                body(p + b, b)

        for g0 in range(T - NBUF + 1, T):
            owait(g0 % NBUF)

    return k(xT, tableP)


def kernel(x, table):
    B, T = x.shape
    xT = x.T.astype(jnp.int32)
    tableP = table.reshape(table.shape[0] // 2, 2 * D)
    out3 = _sc_embed(xT, tableP, T, B)
    return jnp.transpose(out3, (2, 0, 1))


# speedup vs baseline: 1.4054x; 1.4054x over previous
import functools
import math

import jax
import jax.numpy as jnp
from jax import lax
from jax.experimental import pallas as pl
from jax.experimental.pallas import tpu as pltpu
from jax.experimental.pallas import tpu_sc as plsc

D = 64
LANES = 128
NBUF = 4
NC, NS = 2, 16
NW = NC * NS
SCALE = math.sqrt(D)


def _sc_pair(tableT, V):
    NFULL = V // LANES
    TAILV = V - NFULL * LANES
    NSTRIPE = -(-NFULL // NW)
    NSTRIPE = -(-NSTRIPE // NBUF) * NBUF
    mesh = plsc.VectorSubcoreMesh(core_axis_name="c", subcore_axis_name="s")

    @functools.partial(
        pl.kernel,
        mesh=mesh,
        out_type=jax.ShapeDtypeStruct((V // 2, LANES), jnp.float32),
        scratch_types=[
            [pltpu.VMEM((D, LANES), jnp.float32) for _ in range(NBUF)],
            [pltpu.VMEM((D, LANES), jnp.float32) for _ in range(NBUF)],
            [pltpu.SemaphoreType.DMA for _ in range(NBUF)],
            [pltpu.SemaphoreType.DMA for _ in range(NBUF)],
        ],
        compiler_params=pltpu.CompilerParams(
            use_tc_tiling_on_sc=True, needs_layout_passes=False),
    )
    def ka(t_hbm, tail_hbm, p_hbm, tbuf, pbuf, isem, osem):
        wid = lax.axis_index("s") * NC + lax.axis_index("c")
        iota = lax.iota(jnp.int32, 16)

        def vbof(p):
            vb = wid + NW * p
            return jnp.where(vb < NFULL, vb, wid)

        def ifire(vb, s):
            pltpu.async_copy(
                t_hbm.at[:, pl.ds(vb * LANES, LANES)], tbuf[s], isem[s])

        def iwait(s):
            pltpu.make_async_copy(
                t_hbm.at[:, pl.ds(0, LANES)], tbuf[s], isem[s]).wait()

        def ofire(vb, s):
            pltpu.async_copy(pbuf[s], p_hbm.at[pl.ds(vb * D, D)], osem[s])

        def owait(s):
            pltpu.make_async_copy(
                t_hbm.at[:, pl.ds(0, LANES)], pbuf[s], osem[s]).wait()

        def trans(s, nh):
            for h in range(nh):

                @plsc.parallel_loop(0, 16)
                def _(d):
                    perm = (iota + d) & 15
                    uv = iota + 16 * h
                    vbase = 2 * iota + 32 * h
                    for half in range(2):
                        vloc = vbase + half
                        for g in range(D // 16):
                            val = plsc.load_gather(
                                tbuf[s], [perm + 16 * g, vloc])
                            plsc.store_scatter(
                                pbuf[s], [uv, perm + (64 * half + 16 * g)],
                                val)

        def body(p, s):
            s1 = (s + 1) % NBUF

            @pl.when(p >= NBUF - 1)
            def _():
                owait(s1)

            @pl.when(p + 1 < NSTRIPE)
            def _():
                ifire(vbof(p + 1), s1)

            iwait(s)
            trans(s, 4)
            ofire(vbof(p), s)

        ifire(vbof(0), 0)

        @pl.loop(0, NSTRIPE, step=NBUF)
        def _(q):
            for b in range(NBUF):
                body(q + b, b)

        for p0 in range(NSTRIPE - NBUF + 1, NSTRIPE):
            owait(p0 % NBUF)

        if TAILV:
            @pl.when(wid == 0)
            def _():
                pltpu.sync_copy(tail_hbm, pbuf[0].at[pl.ds(0, TAILV // 2)])
                pltpu.sync_copy(
                    pbuf[0].at[pl.ds(0, TAILV // 2)],
                    p_hbm.at[pl.ds(NFULL * D, TAILV // 2)])

    return ka


def _sc_embed(xT, tableP, T, B):
    mesh = plsc.VectorSubcoreMesh(core_axis_name="c", subcore_axis_name="s")

    @functools.partial(
        pl.kernel,
        mesh=mesh,
        out_type=jax.ShapeDtypeStruct((T, D, B), jnp.float32),
        scratch_types=[
            pltpu.VMEM((T, LANES), jnp.int32),
            [pltpu.VMEM((LANES, LANES), jnp.float32) for _ in range(NBUF)],
            [pltpu.VMEM((D, LANES), jnp.float32) for _ in range(NBUF)],
            [pltpu.VMEM((LANES,), jnp.int32) for _ in range(NBUF)],
            [pltpu.VMEM((LANES,), jnp.int32) for _ in range(NBUF)],
            [pltpu.SemaphoreType.DMA for _ in range(NBUF)],
            [pltpu.SemaphoreType.DMA for _ in range(NBUF)],
        ],
        compiler_params=pltpu.CompilerParams(
            use_tc_tiling_on_sc=True, needs_layout_passes=False),
    )
    def k(x_hbm, table_hbm, out_hbm, idxv, rows, obuf, gidx, hbuf, gsem, osem):
        wid = lax.axis_index("s") * NC + lax.axis_index("c")
        b0 = wid * LANES
        iota = lax.iota(jnp.int32, 16)

        def prep(t, s):
            for c in range(LANES // 16):
                v = idxv[t, pl.ds(c * 16, 16)]
                gidx[s][pl.ds(c * 16, 16)] = v >> 1
                hbuf[s][pl.ds(c * 16, 16)] = (v & 1) << 6

        def gfire(s):
            pltpu.async_copy(table_hbm.at[gidx[s]], rows[s], gsem[s])

        def gwait(s):
            pltpu.make_async_copy(
                table_hbm.at[pl.ds(0, LANES)], rows[s], gsem[s]).wait()

        def ofire(t, s):
            pltpu.async_copy(
                obuf[s], out_hbm.at[t, :, pl.ds(b0, LANES)], osem[s])

        def owait(s):
            pltpu.make_async_copy(
                out_hbm.at[0, :, pl.ds(0, LANES)], obuf[s], osem[s]).wait()

        def transpose_scale(s):
            for h in range(LANES // 16):

                @plsc.parallel_loop(0, 16)
                def _(d):
                    perm = (iota + d) & 15
                    rd = perm + 16 * h
                    hv_d = plsc.load_gather(hbuf[s], [rd])
                    colb = hv_d + iota
                    for g in range(D // 16):
                        val = plsc.load_gather(rows[s], [rd, colb + 16 * g])
                        plsc.store_scatter(
                            obuf[s], [iota + 16 * g, rd], val * SCALE)

        def body(g, s):
            s1 = (s + 1) % NBUF

            @pl.when(g >= NBUF - 1)
            def _():
                owait(s1)

            @pl.when(g + 1 < T)
            def _():
                prep(g + 1, s1)
                gfire(s1)

            gwait(s)
            transpose_scale(s)
            ofire(g, s)

        pltpu.sync_copy(x_hbm.at[:, pl.ds(b0, LANES)], idxv)
        prep(0, 0)
        gfire(0)

        @pl.loop(0, T, step=NBUF)
        def _(p):
            for b in range(NBUF):
                body(p + b, b)

        for g0 in range(T - NBUF + 1, T):
            owait(g0 % NBUF)

    return k(xT, tableP)


def kernel(x, table):
    B, T = x.shape
    xT = x.T.astype(jnp.int32)
    tableT = table.T
    V = table.shape[0]
    ntail = V % LANES
    tailP = table[V - ntail:].reshape(ntail // 2, 2 * D)
    tableP = _sc_pair(tableT, V)(tableT, tailP)
    out3 = _sc_embed(xT, tableP, T, B)
    return jnp.transpose(out3, (2, 0, 1))


# speedup vs baseline: 1.7234x; 1.2262x over previous
import functools
import math

import jax
import jax.numpy as jnp
from jax import lax
from jax.experimental import pallas as pl
from jax.experimental.pallas import tpu as pltpu
from jax.experimental.pallas import tpu_sc as plsc

D = 64
LANES = 128
NBUF = 4
NC, NS = 2, 16
NW = NC * NS
SCALE = math.sqrt(D)


def _sc_pair(tableT, V):
    NFULL = V // LANES
    TAILV = V - NFULL * LANES
    NSTRIPE = -(-NFULL // NW)
    NSTRIPE = -(-NSTRIPE // NBUF) * NBUF
    mesh = plsc.VectorSubcoreMesh(core_axis_name="c", subcore_axis_name="s")

    @functools.partial(
        pl.kernel,
        mesh=mesh,
        out_type=jax.ShapeDtypeStruct((V // 2, LANES), jnp.float32),
        scratch_types=[
            [pltpu.VMEM((D, LANES), jnp.float32) for _ in range(NBUF)],
            [pltpu.VMEM((D, LANES), jnp.float32) for _ in range(NBUF)],
            [pltpu.SemaphoreType.DMA for _ in range(NBUF)],
            [pltpu.SemaphoreType.DMA for _ in range(NBUF)],
        ],
        compiler_params=pltpu.CompilerParams(
            use_tc_tiling_on_sc=True, needs_layout_passes=False),
    )
    def ka(t_hbm, tail_hbm, p_hbm, tbuf, pbuf, isem, osem):
        wid = lax.axis_index("s") * NC + lax.axis_index("c")
        iota = lax.iota(jnp.int32, 16)

        def vbof(p):
            vb = wid + NW * p
            return jnp.where(vb < NFULL, vb, wid)

        def ifire(vb, s):
            pltpu.async_copy(
                t_hbm.at[:, pl.ds(vb * LANES, LANES)], tbuf[s], isem[s])

        def iwait(s):
            pltpu.make_async_copy(
                t_hbm.at[:, pl.ds(0, LANES)], tbuf[s], isem[s]).wait()

        def ofire(vb, s):
            pltpu.async_copy(pbuf[s], p_hbm.at[pl.ds(vb * D, D)], osem[s])

        def owait(s):
            pltpu.make_async_copy(
                t_hbm.at[:, pl.ds(0, LANES)], pbuf[s], osem[s]).wait()

        zero = iota & 0

        def trans(s, nh):
            for h in range(nh):

                @plsc.parallel_loop(0, 16)
                def _(d):
                    perm = (iota + d) & 15
                    sbase = (perm << 7) + (2 * iota + 32 * h)
                    dbase = (iota << 7) + perm + 2048 * h
                    for half in range(2):
                        for g in range(D // 16):
                            val = plsc.load_gather(
                                tbuf[s], [zero, sbase + (2048 * g + half)])
                            plsc.store_scatter(
                                pbuf[s],
                                [zero, dbase + (64 * half + 16 * g)], val)

        def body(p, s):
            s2 = (s + 2) % NBUF

            @pl.when(p >= NBUF - 2)
            def _():
                owait(s2)

            @pl.when(p + 2 < NSTRIPE)
            def _():
                ifire(vbof(p + 2), s2)

            iwait(s)
            trans(s, 4)
            ofire(vbof(p), s)

        ifire(vbof(0), 0)
        ifire(vbof(1), 1)

        @pl.loop(0, NSTRIPE, step=NBUF)
        def _(q):
            for b in range(NBUF):
                body(q + b, b)

        for p0 in range(NSTRIPE - 2, NSTRIPE):
            owait(p0 % NBUF)

        if TAILV:
            @pl.when(wid == 0)
            def _():
                pltpu.sync_copy(tail_hbm, pbuf[0].at[pl.ds(0, TAILV // 2)])
                pltpu.sync_copy(
                    pbuf[0].at[pl.ds(0, TAILV // 2)],
                    p_hbm.at[pl.ds(NFULL * D, TAILV // 2)])

    return ka


def _sc_embed(xT, tableP, T, B):
    mesh = plsc.VectorSubcoreMesh(core_axis_name="c", subcore_axis_name="s")

    @functools.partial(
        pl.kernel,
        mesh=mesh,
        out_type=jax.ShapeDtypeStruct((T, D, B), jnp.float32),
        scratch_types=[
            pltpu.VMEM((T, LANES), jnp.int32),
            [pltpu.VMEM((LANES, LANES), jnp.float32) for _ in range(NBUF)],
            [pltpu.VMEM((D, LANES), jnp.float32) for _ in range(NBUF)],
            [pltpu.VMEM((LANES,), jnp.int32) for _ in range(NBUF)],
            [pltpu.VMEM((LANES,), jnp.int32) for _ in range(NBUF)],
            [pltpu.SemaphoreType.DMA for _ in range(NBUF)],
            [pltpu.SemaphoreType.DMA for _ in range(NBUF)],
        ],
        compiler_params=pltpu.CompilerParams(
            use_tc_tiling_on_sc=True, needs_layout_passes=False),
    )
    def k(x_hbm, table_hbm, out_hbm, idxv, rows, obuf, gidx, hbuf, gsem, osem):
        wid = lax.axis_index("s") * NC + lax.axis_index("c")
        b0 = wid * LANES
        iota = lax.iota(jnp.int32, 16)

        def prep(t, s):
            for c in range(LANES // 16):
                v = idxv[t, pl.ds(c * 16, 16)]
                gidx[s][pl.ds(c * 16, 16)] = v >> 1
                hbuf[s][pl.ds(c * 16, 16)] = (v & 1) << 6

        def gfire(s):
            pltpu.async_copy(table_hbm.at[gidx[s]], rows[s], gsem[s])

        def gwait(s):
            pltpu.make_async_copy(
                table_hbm.at[pl.ds(0, LANES)], rows[s], gsem[s]).wait()

        def ofire(t, s):
            pltpu.async_copy(
                obuf[s], out_hbm.at[t, :, pl.ds(b0, LANES)], osem[s])

        def owait(s):
            pltpu.make_async_copy(
                out_hbm.at[0, :, pl.ds(0, LANES)], obuf[s], osem[s]).wait()

        zero = iota & 0

        def transpose_scale(s):
            for h in range(LANES // 16):

                @plsc.parallel_loop(0, 16)
                def _(d):
                    perm = (iota + d) & 15
                    rd = perm + 16 * h
                    hv_d = plsc.load_gather(hbuf[s], [rd])
                    sbase = (rd << 7) + hv_d + iota
                    dbase = (iota << 7) + rd
                    for g in range(D // 16):
                        val = plsc.load_gather(
                            rows[s], [zero, sbase + 16 * g])
                        plsc.store_scatter(
                            obuf[s], [zero, dbase + 2048 * g], val * SCALE)

        def body(g, s):
            s2 = (s + 2) % NBUF

            @pl.when(g >= NBUF - 2)
            def _():
                owait(s2)

            @pl.when(g + 2 < T)
            def _():
                prep(g + 2, s2)
                gfire(s2)

            gwait(s)
            transpose_scale(s)
            ofire(g, s)

        pltpu.sync_copy(x_hbm.at[:, pl.ds(b0, LANES)], idxv)
        prep(0, 0)
        gfire(0)
        prep(1, 1)
        gfire(1)

        @pl.loop(0, T, step=NBUF)
        def _(p):
            for b in range(NBUF):
                body(p + b, b)

        for g0 in range(T - 2, T):
            owait(g0 % NBUF)

    return k(xT, tableP)


def kernel(x, table):
    B, T = x.shape
    xT = x.T.astype(jnp.int32)
    tableT = table.T
    V = table.shape[0]
    ntail = V % LANES
    tailP = table[V - ntail:].reshape(ntail // 2, 2 * D)
    tableP = _sc_pair(tableT, V)(tableT, tailP)
    out3 = _sc_embed(xT, tableP, T, B)
    return jnp.transpose(out3, (2, 0, 1))
